# baseline (speedup 1.0000x reference)
import functools

import jax
import jax.numpy as jnp
from jax.experimental import pallas as pl

NBLK = 2048
QB = 64
C = 96


def _enc_cand(cx, wl, bl, wk, bk, ck, cn):
    h = jax.lax.dot_general(cx[...], wl[...], (((1,), (1,)), ((), ())),
                            preferred_element_type=jnp.float32) + bl[...]
    k = jax.lax.dot_general(h, wk[...], (((1,), (1,)), ((), ())),
                            preferred_element_type=jnp.float32) + bk[...]
    ck[...] = k
    cn[...] = jnp.sum(k * k, axis=1, keepdims=True).reshape(1, -1)


def _enc_query(xn, wl, bl, wk, bk, x1, k):
    h = jax.lax.dot_general(xn[...], wl[...], (((1,), (1,)), ((), ())),
                            preferred_element_type=jnp.float32) + bl[...]
    x1[...] = h
    k[...] = jax.lax.dot_general(h, wk[...], (((1,), (1,)), ((), ())),
                                 preferred_element_type=jnp.float32) + bk[...]


def _scores(n_real, k, ck, cn, out):
    j = pl.program_id(0)
    s = 2.0 * jax.lax.dot_general(k[...], ck[...], (((1,), (1,)), ((), ())),
                                  preferred_element_type=jnp.float32) - cn[...]
    col = j * NBLK + jax.lax.broadcasted_iota(jnp.int32, s.shape, 1)
    out[...] = jnp.where(col < n_real, s, -3.0e38)


def _softmax(sv, out):
    s = sv[...]
    m = jnp.max(s, axis=-1, keepdims=True)
    e = jnp.exp(s - m)
    out[...] = e / jnp.sum(e, axis=-1, keepdims=True)


def _tail(x, kf, ckf, pT, yf, wle, ble, wt1, bt1, wt2, ln1w, ln1b,
          wb1, bb1, wb2, bb2, lnhw, lnhb, wh, bh, out):
    diff = kf[...] - ckf[...]
    t1 = jax.lax.dot_general(diff, wt1[...], (((1,), (1,)), ((), ())),
                             preferred_element_type=jnp.float32) + bt1[...]
    t1 = jnp.maximum(t1, 0.0)
    t2 = jax.lax.dot_general(t1, wt2[...], (((1,), (1,)), ((), ())),
                             preferred_element_type=jnp.float32)
    ye = jax.lax.dot_general(yf[...], wle[...], (((1,), (0,)), ((), ())),
                             preferred_element_type=jnp.float32)
    values = ye + ble[...] + t2

    rows = jax.lax.broadcasted_iota(jnp.int32, (QB, QB * C), 0)
    colq = jax.lax.broadcasted_iota(jnp.int32, (QB, QB * C), 1) // C
    sel = jnp.where(rows == colq, pT[...], 0.0)
    ctx = jax.lax.dot_general(sel, values, (((1,), (0,)), ((), ())),
                              preferred_element_type=jnp.float32)

    xv = x[...] + ctx
    mu = jnp.mean(xv, axis=-1, keepdims=True)
    var = jnp.mean((xv - mu) ** 2, axis=-1, keepdims=True)
    h = (xv - mu) * jax.lax.rsqrt(var + 1e-5) * ln1w[...] + ln1b[...]
    h = jax.lax.dot_general(h, wb1[...], (((1,), (1,)), ((), ())),
                            preferred_element_type=jnp.float32) + bb1[...]
    h = jnp.maximum(h, 0.0)
    h = jax.lax.dot_general(h, wb2[...], (((1,), (1,)), ((), ())),
                            preferred_element_type=jnp.float32) + bb2[...]
    xv = xv + h
    mu = jnp.mean(xv, axis=-1, keepdims=True)
    var = jnp.mean((xv - mu) ** 2, axis=-1, keepdims=True)
    h = (xv - mu) * jax.lax.rsqrt(var + 1e-5) * lnhw[...] + lnhb[...]
    h = jnp.maximum(h, 0.0)
    out[...] = jnp.sum(h * wh[...], axis=-1, keepdims=True) + bh[0, 0]


def _whole(shape):
    return pl.BlockSpec(shape, lambda *_: tuple(0 for _ in shape))


def kernel(x_num, candidate_x_num, candidate_y, W_lin, b_lin, W_K, b_K,
           W_le, b_le, W_t1, b_t1, W_t2, ln1_w, ln1_b, W_b1, b_b1,
           W_b2, b_b2, lnh_w, lnh_b, W_h, b_h, context_size):
    Q, D_IN = x_num.shape
    N = candidate_x_num.shape[0]
    D = W_lin.shape[0]
    npad = ((N + NBLK - 1) // NBLK) * NBLK
    gn = npad // NBLK
    cx = jnp.pad(candidate_x_num, ((0, npad - N), (0, 0)))
    cy = jnp.pad(candidate_y, (0, npad - N))

    ck, cn = pl.pallas_call(
        _enc_cand,
        grid=(gn,),
        in_specs=[pl.BlockSpec((NBLK, D_IN), lambda i: (i, 0)),
                  _whole(W_lin.shape), _whole((1, D)),
                  _whole(W_K.shape), _whole((1, D))],
        out_specs=[pl.BlockSpec((NBLK, D), lambda i: (i, 0)),
                   pl.BlockSpec((1, NBLK), lambda i: (0, i))],
        out_shape=[jax.ShapeDtypeStruct((npad, D), jnp.float32),
                   jax.ShapeDtypeStruct((1, npad), jnp.float32)],
    )(cx, W_lin, b_lin.reshape(1, D), W_K, b_K.reshape(1, D))

    x1, k = pl.pallas_call(
        _enc_query,
        in_specs=[_whole((Q, D_IN)), _whole(W_lin.shape), _whole((1, D)),
                  _whole(W_K.shape), _whole((1, D))],
        out_specs=[_whole((Q, D)), _whole((Q, D))],
        out_shape=[jax.ShapeDtypeStruct((Q, D), jnp.float32),
                   jax.ShapeDtypeStruct((Q, D), jnp.float32)],
    )(x_num, W_lin, b_lin.reshape(1, D), W_K, b_K.reshape(1, D))

    scores = pl.pallas_call(
        functools.partial(_scores, N),
        grid=(gn,),
        in_specs=[_whole((Q, D)),
                  pl.BlockSpec((NBLK, D), lambda j: (j, 0)),
                  pl.BlockSpec((1, NBLK), lambda j: (0, j))],
        out_specs=pl.BlockSpec((Q, NBLK), lambda j: (0, j)),
        out_shape=jax.ShapeDtypeStruct((Q, npad), jnp.float32),
    )(k, ck, cn)

    svals, idx = jax.lax.top_k(scores, C)
    idxf = idx.reshape(-1)

    probs = pl.pallas_call(
        _softmax,
        in_specs=[_whole((Q, C))],
        out_specs=_whole((Q, C)),
        out_shape=jax.ShapeDtypeStruct((Q, C), jnp.float32),
    )(svals)

    ckf = ck[idxf]
    yf = cy[idxf].reshape(Q * C, 1)
    kf = jnp.repeat(k, C, axis=0)
    pT = probs.reshape(1, Q * C)

    out = pl.pallas_call(
        _tail,
        grid=(Q // QB,),
        in_specs=[pl.BlockSpec((QB, D), lambda i: (i, 0)),
                  pl.BlockSpec((QB * C, D), lambda i: (i, 0)),
                  pl.BlockSpec((QB * C, D), lambda i: (i, 0)),
                  pl.BlockSpec((1, QB * C), lambda i: (0, i)),
                  pl.BlockSpec((QB * C, 1), lambda i: (i, 0)),
                  _whole((1, D)), _whole((1, D)),
                  _whole(W_t1.shape), _whole((1, W_t1.shape[0])),
                  _whole(W_t2.shape),
                  _whole((1, D)), _whole((1, D)),
                  _whole(W_b1.shape), _whole((1, W_b1.shape[0])),
                  _whole(W_b2.shape), _whole((1, D)),
                  _whole((1, D)), _whole((1, D)),
                  _whole(W_h.shape), _whole((1, 1))],
        out_specs=pl.BlockSpec((QB, 1), lambda i: (i, 0)),
        out_shape=jax.ShapeDtypeStruct((Q, 1), jnp.float32),
    )(x1, kf, ckf, pT, yf,
      W_le.reshape(1, D), b_le.reshape(1, D),
      W_t1, b_t1.reshape(1, -1), W_t2,
      ln1_w.reshape(1, D), ln1_b.reshape(1, D),
      W_b1, b_b1.reshape(1, -1), W_b2, b_b2.reshape(1, D),
      lnh_w.reshape(1, D), lnh_b.reshape(1, D), W_h, b_h.reshape(1, 1))
    return out


# speedup vs baseline: 3.9837x; 3.9837x over previous
import functools

import jax
import jax.numpy as jnp
from jax import lax
from jax.experimental import pallas as pl
from jax.experimental.pallas import tpu as pltpu
from jax.experimental.pallas import tpu_sc as plsc

NBLK = 2048
QB = 64
QBT = 32
C = 96
AW = 128
NEG = -3.0e38


def _enc_cand(cx, cy, wl, bl, wk, bk, aug, cn):
    h = lax.dot_general(cx[...], wl[...], (((1,), (1,)), ((), ())),
                        preferred_element_type=jnp.float32) + bl[...]
    k = lax.dot_general(h, wk[...], (((1,), (1,)), ((), ())),
                        preferred_element_type=jnp.float32) + bk[...]
    D = k.shape[1]
    col = lax.broadcasted_iota(jnp.int32, (k.shape[0], AW), 1)
    kwide = jnp.pad(k, ((0, 0), (0, AW - D)))
    ywide = cy[...] * (col == D).astype(jnp.float32)
    aug[...] = jnp.where(col < D, kwide, ywide)
    cn[...] = jnp.sum(k * k, axis=1, keepdims=True).reshape(1, -1)


def _enc_query(xn, wl, bl, wk, bk, x1, k):
    h = lax.dot_general(xn[...], wl[...], (((1,), (1,)), ((), ())),
                        preferred_element_type=jnp.float32) + bl[...]
    x1[...] = h
    k[...] = lax.dot_general(h, wk[...], (((1,), (1,)), ((), ())),
                             preferred_element_type=jnp.float32) + bk[...]


def _scores(n_real, k, aug, cn, out):
    j = pl.program_id(0)
    ck = aug[...][:, :k.shape[1]]
    s = 2.0 * lax.dot_general(k[...], ck, (((1,), (1,)), ((), ())),
                              preferred_element_type=jnp.float32) - cn[...]
    col = j * NBLK + lax.broadcasted_iota(jnp.int32, s.shape, 1)
    out[...] = jnp.where(col < n_real, s, NEG)


def _thresh(sv, out):
    s = sv[...]
    key = lax.bitcast_convert_type(s, jnp.int32)
    m = jnp.where(key >= 0, key, key ^ 0x7FFFFFFF)
    imin = jnp.int32(-2147483648)
    imax = jnp.int32(2147483647)
    cnt0 = jnp.sum((m >= 0).astype(jnp.int32), axis=1, keepdims=True)
    pos = cnt0 >= C
    lo = jnp.where(pos, jnp.int32(0), imin)
    hi = jnp.where(pos, imax, jnp.int32(-1))

    def body(_, lh):
        lo, hi = lh
        mid = hi - ((hi - lo) >> 1)
        cnt = jnp.sum((m >= mid).astype(jnp.int32), axis=1, keepdims=True)
        ge = cnt >= C
        return jnp.where(ge, mid, lo), jnp.where(ge, hi, mid - 1)

    lo, hi = lax.fori_loop(0, 32, body, (lo, hi))
    tkey = jnp.where(lo >= 0, lo, lo ^ 0x7FFFFFFF)
    tf = lax.bitcast_convert_type(tkey, jnp.float32)
    out[...] = jnp.broadcast_to(tf, (tf.shape[0], 128))


def _softmax(sv, out):
    s = sv[...]
    mx = jnp.max(s, axis=-1, keepdims=True)
    e = jnp.exp(s - mx)
    out[...] = e / jnp.sum(e, axis=-1, keepdims=True)


def _tail(x, kf, augf, pT, wle, ble, wt1, bt1, wt2, ln1w, ln1b,
          wb1, bb1, wb2, bb2, lnhw, lnhb, wh, bh, out):
    D = x.shape[1]
    ckf = augf[...][:, :D]
    yf = augf[...][:, D:D + 1]
    diff = kf[...] - ckf
    t1 = lax.dot_general(diff, wt1[...], (((1,), (1,)), ((), ())),
                         preferred_element_type=jnp.float32) + bt1[...]
    t1 = jnp.maximum(t1, 0.0)
    t2 = lax.dot_general(t1, wt2[...], (((1,), (1,)), ((), ())),
                         preferred_element_type=jnp.float32)
    ye = lax.dot_general(yf, wle[...], (((1,), (0,)), ((), ())),
                         preferred_element_type=jnp.float32)
    values = ye + ble[...] + t2

    rows = lax.broadcasted_iota(jnp.int32, (QB, QB * C), 0)
    colq = lax.broadcasted_iota(jnp.int32, (QB, QB * C), 1) // C
    sel = jnp.where(rows == colq, pT[...], 0.0)
    ctx = lax.dot_general(sel, values, (((1,), (0,)), ((), ())),
                          preferred_element_type=jnp.float32)

    xv = x[...] + ctx
    mu = jnp.mean(xv, axis=-1, keepdims=True)
    var = jnp.mean((xv - mu) ** 2, axis=-1, keepdims=True)
    h = (xv - mu) * lax.rsqrt(var + 1e-5) * ln1w[...] + ln1b[...]
    h = lax.dot_general(h, wb1[...], (((1,), (1,)), ((), ())),
                        preferred_element_type=jnp.float32) + bb1[...]
    h = jnp.maximum(h, 0.0)
    h = lax.dot_general(h, wb2[...], (((1,), (1,)), ((), ())),
                        preferred_element_type=jnp.float32) + bb2[...]
    xv = xv + h
    mu = jnp.mean(xv, axis=-1, keepdims=True)
    var = jnp.mean((xv - mu) ** 2, axis=-1, keepdims=True)
    h = (xv - mu) * lax.rsqrt(var + 1e-5) * lnhw[...] + lnhb[...]
    h = jnp.maximum(h, 0.0)
    out[...] = jnp.sum(h * wh[...], axis=-1, keepdims=True) + bh[0, 0]


def _whole(shape):
    return pl.BlockSpec(shape, lambda *_: tuple(0 for _ in shape))


def _make_select(Q, npad, nsteps):
    NW = 32
    qpw = Q // NW
    mesh = plsc.VectorSubcoreMesh(core_axis_name="c", subcore_axis_name="s")

    @functools.partial(
        pl.kernel, mesh=mesh,
        compiler_params=pltpu.CompilerParams(needs_layout_passes=False),
        out_type=[jax.ShapeDtypeStruct((Q * C, AW), jnp.float32),
                  jax.ShapeDtypeStruct((Q, C), jnp.float32)],
        scratch_types=[pltpu.VMEM((npad,), jnp.float32),
                       pltpu.VMEM((16,), jnp.float32),
                       pltpu.VMEM((112,), jnp.int32),
                       pltpu.VMEM((112,), jnp.int32),
                       pltpu.VMEM((C,), jnp.int32),
                       pltpu.VMEM((C,), jnp.float32),
                       pltpu.VMEM((C, AW), jnp.float32),
                       pltpu.SemaphoreType.DMA],
    )
    def sc_select(scores_hbm, thr_hbm, aug_hbm, out_hbm, ssel_hbm,
                  row_v, thr_v, bufa, bufb, idx_v, ssel_v, rows_v, sem):
        wid = lax.axis_index("s") * 2 + lax.axis_index("c")
        iota16 = lax.broadcasted_iota(jnp.int32, (16,), 0)

        def per_query(qi, _):
            q = wid * qpw + qi
            pltpu.sync_copy(scores_hbm.at[q], row_v)
            pltpu.sync_copy(thr_hbm.at[q, pl.ds(0, 16)], thr_v)
            tvec = thr_v[...]

            def step(i, carry):
                ca, cb = carry
                vec = row_v[pl.ds(i * 16, 16)]
                colv = iota16 + i * 16
                m_gt = vec > tvec
                pref_a = jnp.cumsum(jnp.where(m_gt, 1, 0))
                posa = jnp.where(m_gt, ca + pref_a - 1, 111)
                plsc.store_scatter(bufa, [posa], colv)
                ca = ca + jnp.max(pref_a)
                m_eq = vec == tvec
                pref_e = jnp.cumsum(jnp.where(m_eq, 1, 0))
                keep = m_eq & ((cb + pref_e) <= C)
                posb = jnp.where(keep, cb + pref_e - 1, 111)
                plsc.store_scatter(bufb, [posb], colv)
                cb = cb + jnp.max(jnp.where(keep, pref_e, 0))
                return ca, cb

            ca, _cb = lax.fori_loop(0, nsteps, step, (jnp.int32(0), jnp.int32(0)))

            for kb in range(C // 16):
                lane = iota16 + kb * 16
                in_a = lane < ca
                ia = plsc.load_gather(bufa, [lane])
                pb = jnp.maximum(lane - ca, 0)
                ib = plsc.load_gather(bufb, [pb])
                iv = jnp.where(in_a, ia, ib)
                idx_v[pl.ds(kb * 16, 16)] = iv
                ssel_v[pl.ds(kb * 16, 16)] = plsc.load_gather(row_v, [iv])

            pltpu.async_copy(aug_hbm.at[idx_v], rows_v, sem).wait()
            pltpu.sync_copy(rows_v, out_hbm.at[pl.ds(q * C, C)])
            pltpu.sync_copy(ssel_v, ssel_hbm.at[q])
            return 0

        lax.fori_loop(0, qpw, per_query, 0)

    return sc_select


def kernel(x_num, candidate_x_num, candidate_y, W_lin, b_lin, W_K, b_K,
           W_le, b_le, W_t1, b_t1, W_t2, ln1_w, ln1_b, W_b1, b_b1,
           W_b2, b_b2, lnh_w, lnh_b, W_h, b_h, context_size):
    Q, D_IN = x_num.shape
    N = candidate_x_num.shape[0]
    D = W_lin.shape[0]
    npad = ((N + NBLK - 1) // NBLK) * NBLK
    gn = npad // NBLK
    cx = jnp.pad(candidate_x_num, ((0, npad - N), (0, 0)))
    cy = jnp.pad(candidate_y, (0, npad - N)).reshape(npad, 1)

    aug, cn = pl.pallas_call(
        _enc_cand,
        grid=(gn,),
        in_specs=[pl.BlockSpec((NBLK, D_IN), lambda i: (i, 0)),
                  pl.BlockSpec((NBLK, 1), lambda i: (i, 0)),
                  _whole(W_lin.shape), _whole((1, D)),
                  _whole(W_K.shape), _whole((1, D))],
        out_specs=[pl.BlockSpec((NBLK, AW), lambda i: (i, 0)),
                   pl.BlockSpec((1, NBLK), lambda i: (0, i))],
        out_shape=[jax.ShapeDtypeStruct((npad, AW), jnp.float32),
                   jax.ShapeDtypeStruct((1, npad), jnp.float32)],
    )(cx, cy, W_lin, b_lin.reshape(1, D), W_K, b_K.reshape(1, D))

    x1, k = pl.pallas_call(
        _enc_query,
        in_specs=[_whole((Q, D_IN)), _whole(W_lin.shape), _whole((1, D)),
                  _whole(W_K.shape), _whole((1, D))],
        out_specs=[_whole((Q, D)), _whole((Q, D))],
        out_shape=[jax.ShapeDtypeStruct((Q, D), jnp.float32),
                   jax.ShapeDtypeStruct((Q, D), jnp.float32)],
    )(x_num, W_lin, b_lin.reshape(1, D), W_K, b_K.reshape(1, D))

    scores = pl.pallas_call(
        functools.partial(_scores, N),
        grid=(gn,),
        in_specs=[_whole((Q, D)),
                  pl.BlockSpec((NBLK, AW), lambda j: (j, 0)),
                  pl.BlockSpec((1, NBLK), lambda j: (0, j))],
        out_specs=pl.BlockSpec((Q, NBLK), lambda j: (0, j)),
        out_shape=jax.ShapeDtypeStruct((Q, npad), jnp.float32),
    )(k, aug, cn)

    thr = pl.pallas_call(
        _thresh,
        grid=(Q // QBT,),
        in_specs=[pl.BlockSpec((QBT, npad), lambda i: (i, 0))],
        out_specs=pl.BlockSpec((QBT, 128), lambda i: (i, 0)),
        out_shape=jax.ShapeDtypeStruct((Q, 128), jnp.float32),
    )(scores)

    augf, ssel = _make_select(Q, npad, npad // 16)(scores, thr, aug)

    probs = pl.pallas_call(
        _softmax,
        in_specs=[_whole((Q, C))],
        out_specs=_whole((Q, C)),
        out_shape=jax.ShapeDtypeStruct((Q, C), jnp.float32),
    )(ssel)

    kf = jnp.repeat(k, C, axis=0)
    pT = probs.reshape(1, Q * C)

    out = pl.pallas_call(
        _tail,
        grid=(Q // QB,),
        in_specs=[pl.BlockSpec((QB, D), lambda i: (i, 0)),
                  pl.BlockSpec((QB * C, D), lambda i: (i, 0)),
                  pl.BlockSpec((QB * C, AW), lambda i: (i, 0)),
                  pl.BlockSpec((1, QB * C), lambda i: (0, i)),
                  _whole((1, D)), _whole((1, D)),
                  _whole(W_t1.shape), _whole((1, W_t1.shape[0])),
                  _whole(W_t2.shape),
                  _whole((1, D)), _whole((1, D)),
                  _whole(W_b1.shape), _whole((1, W_b1.shape[0])),
                  _whole(W_b2.shape), _whole((1, D)),
                  _whole((1, D)), _whole((1, D)),
                  _whole(W_h.shape), _whole((1, 1))],
        out_specs=pl.BlockSpec((QB, 1), lambda i: (i, 0)),
        out_shape=jax.ShapeDtypeStruct((Q, 1), jnp.float32),
    )(x1, kf, augf, pT,
      W_le.reshape(1, D), b_le.reshape(1, D),
      W_t1, b_t1.reshape(1, -1), W_t2,
      ln1_w.reshape(1, D), ln1_b.reshape(1, D),
      W_b1, b_b1.reshape(1, -1), W_b2, b_b2.reshape(1, D),
      lnh_w.reshape(1, D), lnh_b.reshape(1, D), W_h, b_h.reshape(1, 1))
    return out


# speedup vs baseline: 7.2872x; 1.8292x over previous
import functools

import jax
import jax.numpy as jnp
from jax import lax
from jax.experimental import pallas as pl
from jax.experimental.pallas import tpu as pltpu
from jax.experimental.pallas import tpu_sc as plsc

NBLK = 2048
QB = 64
QBT = 32
C = 96
AW = 128
NEG = -3.0e38


def _enc_cand(cx, cy, wl, bl, wk, bk, aug, cn):
    h = lax.dot_general(cx[...], wl[...], (((1,), (1,)), ((), ())),
                        preferred_element_type=jnp.float32) + bl[...]
    k = lax.dot_general(h, wk[...], (((1,), (1,)), ((), ())),
                        preferred_element_type=jnp.float32) + bk[...]
    D = k.shape[1]
    col = lax.broadcasted_iota(jnp.int32, (k.shape[0], AW), 1)
    kwide = jnp.pad(k, ((0, 0), (0, AW - D)))
    ywide = cy[...] * (col == D).astype(jnp.float32)
    aug[...] = jnp.where(col < D, kwide, ywide)
    cn[...] = jnp.sum(k * k, axis=1, keepdims=True).reshape(1, -1)


def _enc_query(xn, wl, bl, wk, bk, x1, k):
    h = lax.dot_general(xn[...], wl[...], (((1,), (1,)), ((), ())),
                        preferred_element_type=jnp.float32) + bl[...]
    x1[...] = h
    k[...] = lax.dot_general(h, wk[...], (((1,), (1,)), ((), ())),
                             preferred_element_type=jnp.float32) + bk[...]


def _scores(n_real, k, aug, cn, out):
    j = pl.program_id(0)
    ck = aug[...][:, :k.shape[1]]
    s = 2.0 * lax.dot_general(k[...], ck, (((1,), (1,)), ((), ())),
                              preferred_element_type=jnp.float32) - cn[...]
    col = j * NBLK + lax.broadcasted_iota(jnp.int32, s.shape, 1)
    out[...] = jnp.where(col < n_real, s, NEG)


def _thresh(sv, out, cmx):
    s = sv[...]
    key = lax.bitcast_convert_type(s, jnp.int32)
    m = jnp.where(key >= 0, key, key ^ 0x7FFFFFFF)
    imin = jnp.int32(-2147483648)
    imax = jnp.int32(2147483647)
    cnt0 = jnp.sum((m >= 0).astype(jnp.int32), axis=1, keepdims=True)
    pos = cnt0 >= C
    lo = jnp.where(pos, jnp.int32(0), imin)
    hi = jnp.where(pos, imax, jnp.int32(-1))

    def body(_, lh):
        lo, hi = lh
        mid = hi - ((hi - lo) >> 1)
        cnt = jnp.sum((m >= mid).astype(jnp.int32), axis=1, keepdims=True)
        ge = cnt >= C
        return jnp.where(ge, mid, lo), jnp.where(ge, hi, mid - 1)

    lo, hi = lax.fori_loop(0, 32, body, (lo, hi))
    tkey = jnp.where(lo >= 0, lo, lo ^ 0x7FFFFFFF)
    tf = lax.bitcast_convert_type(tkey, jnp.float32)
    out[...] = jnp.broadcast_to(tf, (tf.shape[0], 128))
    nch = s.shape[1] // 128
    cmx[...] = jnp.max(s.reshape(s.shape[0], nch, 128), axis=2)


def _softmax(sv, out):
    s = sv[...]
    mx = jnp.max(s, axis=-1, keepdims=True)
    e = jnp.exp(s - mx)
    out[...] = e / jnp.sum(e, axis=-1, keepdims=True)


def _tail(x, kf, augf, pT, wle, ble, wt1, bt1, wt2, ln1w, ln1b,
          wb1, bb1, wb2, bb2, lnhw, lnhb, wh, bh, out):
    D = x.shape[1]
    ckf = augf[...][:, :D]
    yf = augf[...][:, D:D + 1]
    diff = kf[...] - ckf
    t1 = lax.dot_general(diff, wt1[...], (((1,), (1,)), ((), ())),
                         preferred_element_type=jnp.float32) + bt1[...]
    t1 = jnp.maximum(t1, 0.0)
    t2 = lax.dot_general(t1, wt2[...], (((1,), (1,)), ((), ())),
                         preferred_element_type=jnp.float32)
    ye = lax.dot_general(yf, wle[...], (((1,), (0,)), ((), ())),
                         preferred_element_type=jnp.float32)
    values = ye + ble[...] + t2

    rows = lax.broadcasted_iota(jnp.int32, (QB, QB * C), 0)
    colq = lax.broadcasted_iota(jnp.int32, (QB, QB * C), 1) // C
    sel = jnp.where(rows == colq, pT[...], 0.0)
    ctx = lax.dot_general(sel, values, (((1,), (0,)), ((), ())),
                          preferred_element_type=jnp.float32)

    xv = x[...] + ctx
    mu = jnp.mean(xv, axis=-1, keepdims=True)
    var = jnp.mean((xv - mu) ** 2, axis=-1, keepdims=True)
    h = (xv - mu) * lax.rsqrt(var + 1e-5) * ln1w[...] + ln1b[...]
    h = lax.dot_general(h, wb1[...], (((1,), (1,)), ((), ())),
                        preferred_element_type=jnp.float32) + bb1[...]
    h = jnp.maximum(h, 0.0)
    h = lax.dot_general(h, wb2[...], (((1,), (1,)), ((), ())),
                        preferred_element_type=jnp.float32) + bb2[...]
    xv = xv + h
    mu = jnp.mean(xv, axis=-1, keepdims=True)
    var = jnp.mean((xv - mu) ** 2, axis=-1, keepdims=True)
    h = (xv - mu) * lax.rsqrt(var + 1e-5) * lnhw[...] + lnhb[...]
    h = jnp.maximum(h, 0.0)
    out[...] = jnp.sum(h * wh[...], axis=-1, keepdims=True) + bh[0, 0]


def _whole(shape):
    return pl.BlockSpec(shape, lambda *_: tuple(0 for _ in shape))


def _make_select(Q, npad, nchunk):
    NW = 32
    qpw = Q // NW
    mesh = plsc.VectorSubcoreMesh(core_axis_name="c", subcore_axis_name="s")

    @functools.partial(
        pl.kernel, mesh=mesh,
        compiler_params=pltpu.CompilerParams(needs_layout_passes=False),
        out_type=[jax.ShapeDtypeStruct((Q * C, AW), jnp.float32),
                  jax.ShapeDtypeStruct((Q, C), jnp.float32)],
        scratch_types=[pltpu.VMEM((npad,), jnp.float32),
                       pltpu.VMEM((16,), jnp.float32),
                       pltpu.VMEM((112,), jnp.int32),
                       pltpu.VMEM((112,), jnp.int32),
                       pltpu.VMEM((C,), jnp.int32),
                       pltpu.VMEM((C,), jnp.float32),
                       pltpu.VMEM((C, AW), jnp.float32),
                       pltpu.VMEM((nchunk,), jnp.float32),
                       pltpu.VMEM((nchunk + 16,), jnp.int32),
                       pltpu.SemaphoreType.DMA],
    )
    def sc_select(scores_hbm, thr_hbm, cmx_hbm, aug_hbm, out_hbm, ssel_hbm,
                  row_v, thr_v, bufa, bufb, idx_v, ssel_v, rows_v,
                  cmx_v, chunkbuf, sem):
        wid = lax.axis_index("s") * 2 + lax.axis_index("c")
        iota16 = lax.broadcasted_iota(jnp.int32, (16,), 0)

        def per_query(qi, _):
            q = wid * qpw + qi
            pltpu.sync_copy(scores_hbm.at[q], row_v)
            pltpu.sync_copy(cmx_hbm.at[q], cmx_v)
            pltpu.sync_copy(thr_hbm.at[q, pl.ds(0, 16)], thr_v)
            tvec = thr_v[...]

            def cstep(i, nc):
                vm = cmx_v[pl.ds(i * 16, 16)]
                hit = vm >= tvec
                pr = jnp.cumsum(jnp.where(hit, 1, 0))
                pos = jnp.where(hit, nc + pr - 1, nchunk + 15)
                plsc.store_scatter(chunkbuf, [pos], iota16 + i * 16)
                return nc + jnp.max(pr)

            nc = lax.fori_loop(0, nchunk // 16, cstep, jnp.int32(0))

            def proc(ci, carry):
                ca, cb = carry
                cid = plsc.load_gather(chunkbuf, [iota16 * 0 + ci])
                base = cid * 128
                for t in range(8):
                    colv = base + t * 16 + iota16
                    vec = plsc.load_gather(row_v, [colv])
                    m_gt = vec > tvec
                    pref_a = jnp.cumsum(jnp.where(m_gt, 1, 0))
                    posa = jnp.where(m_gt, ca + pref_a - 1, 111)
                    plsc.store_scatter(bufa, [posa], colv)
                    ca = ca + jnp.max(pref_a)
                    m_eq = vec == tvec
                    pref_e = jnp.cumsum(jnp.where(m_eq, 1, 0))
                    keep = m_eq & ((cb + pref_e) <= C)
                    posb = jnp.where(keep, cb + pref_e - 1, 111)
                    plsc.store_scatter(bufb, [posb], colv)
                    cb = cb + jnp.max(jnp.where(keep, pref_e, 0))
                return ca, cb

            ca, _cb = lax.fori_loop(0, nc, proc, (jnp.int32(0), jnp.int32(0)))

            for kb in range(C // 16):
                lane = iota16 + kb * 16
                in_a = lane < ca
                ia = plsc.load_gather(bufa, [lane])
                pb = jnp.maximum(lane - ca, 0)
                ib = plsc.load_gather(bufb, [pb])
                iv = jnp.where(in_a, ia, ib)
                idx_v[pl.ds(kb * 16, 16)] = iv
                ssel_v[pl.ds(kb * 16, 16)] = plsc.load_gather(row_v, [iv])

            pltpu.async_copy(aug_hbm.at[idx_v], rows_v, sem).wait()
            pltpu.sync_copy(rows_v, out_hbm.at[pl.ds(q * C, C)])
            pltpu.sync_copy(ssel_v, ssel_hbm.at[q])
            return 0

        lax.fori_loop(0, qpw, per_query, 0)

    return sc_select


def kernel(x_num, candidate_x_num, candidate_y, W_lin, b_lin, W_K, b_K,
           W_le, b_le, W_t1, b_t1, W_t2, ln1_w, ln1_b, W_b1, b_b1,
           W_b2, b_b2, lnh_w, lnh_b, W_h, b_h, context_size):
    Q, D_IN = x_num.shape
    N = candidate_x_num.shape[0]
    D = W_lin.shape[0]
    npad = ((N + NBLK - 1) // NBLK) * NBLK
    gn = npad // NBLK
    cx = jnp.pad(candidate_x_num, ((0, npad - N), (0, 0)))
    cy = jnp.pad(candidate_y, (0, npad - N)).reshape(npad, 1)

    aug, cn = pl.pallas_call(
        _enc_cand,
        grid=(gn,),
        in_specs=[pl.BlockSpec((NBLK, D_IN), lambda i: (i, 0)),
                  pl.BlockSpec((NBLK, 1), lambda i: (i, 0)),
                  _whole(W_lin.shape), _whole((1, D)),
                  _whole(W_K.shape), _whole((1, D))],
        out_specs=[pl.BlockSpec((NBLK, AW), lambda i: (i, 0)),
                   pl.BlockSpec((1, NBLK), lambda i: (0, i))],
        out_shape=[jax.ShapeDtypeStruct((npad, AW), jnp.float32),
                   jax.ShapeDtypeStruct((1, npad), jnp.float32)],
    )(cx, cy, W_lin, b_lin.reshape(1, D), W_K, b_K.reshape(1, D))

    x1, k = pl.pallas_call(
        _enc_query,
        in_specs=[_whole((Q, D_IN)), _whole(W_lin.shape), _whole((1, D)),
                  _whole(W_K.shape), _whole((1, D))],
        out_specs=[_whole((Q, D)), _whole((Q, D))],
        out_shape=[jax.ShapeDtypeStruct((Q, D), jnp.float32),
                   jax.ShapeDtypeStruct((Q, D), jnp.float32)],
    )(x_num, W_lin, b_lin.reshape(1, D), W_K, b_K.reshape(1, D))

    nchunk = npad // 128
    scores = pl.pallas_call(
        functools.partial(_scores, N),
        grid=(gn,),
        in_specs=[_whole((Q, D)),
                  pl.BlockSpec((NBLK, AW), lambda j: (j, 0)),
                  pl.BlockSpec((1, NBLK), lambda j: (0, j))],
        out_specs=pl.BlockSpec((Q, NBLK), lambda j: (0, j)),
        out_shape=jax.ShapeDtypeStruct((Q, npad), jnp.float32),
    )(k, aug, cn)

    thr, cmax = pl.pallas_call(
        _thresh,
        grid=(Q // QBT,),
        in_specs=[pl.BlockSpec((QBT, npad), lambda i: (i, 0))],
        out_specs=[pl.BlockSpec((QBT, 128), lambda i: (i, 0)),
                   pl.BlockSpec((QBT, nchunk), lambda i: (i, 0))],
        out_shape=[jax.ShapeDtypeStruct((Q, 128), jnp.float32),
                   jax.ShapeDtypeStruct((Q, nchunk), jnp.float32)],
    )(scores)

    augf, ssel = _make_select(Q, npad, nchunk)(scores, thr, cmax, aug)

    probs = pl.pallas_call(
        _softmax,
        in_specs=[_whole((Q, C))],
        out_specs=_whole((Q, C)),
        out_shape=jax.ShapeDtypeStruct((Q, C), jnp.float32),
    )(ssel)

    kf = jnp.repeat(k, C, axis=0)
    pT = probs.reshape(1, Q * C)

    out = pl.pallas_call(
        _tail,
        grid=(Q // QB,),
        in_specs=[pl.BlockSpec((QB, D), lambda i: (i, 0)),
                  pl.BlockSpec((QB * C, D), lambda i: (i, 0)),
                  pl.BlockSpec((QB * C, AW), lambda i: (i, 0)),
                  pl.BlockSpec((1, QB * C), lambda i: (0, i)),
                  _whole((1, D)), _whole((1, D)),
                  _whole(W_t1.shape), _whole((1, W_t1.shape[0])),
                  _whole(W_t2.shape),
                  _whole((1, D)), _whole((1, D)),
                  _whole(W_b1.shape), _whole((1, W_b1.shape[0])),
                  _whole(W_b2.shape), _whole((1, D)),
                  _whole((1, D)), _whole((1, D)),
                  _whole(W_h.shape), _whole((1, 1))],
        out_specs=pl.BlockSpec((QB, 1), lambda i: (i, 0)),
        out_shape=jax.ShapeDtypeStruct((Q, 1), jnp.float32),
    )(x1, kf, augf, pT,
      W_le.reshape(1, D), b_le.reshape(1, D),
      W_t1, b_t1.reshape(1, -1), W_t2,
      ln1_w.reshape(1, D), ln1_b.reshape(1, D),
      W_b1, b_b1.reshape(1, -1), W_b2, b_b2.reshape(1, D),
      lnh_w.reshape(1, D), lnh_b.reshape(1, D), W_h, b_h.reshape(1, 1))
    return out
